# baseline (speedup 1.0000x reference)
import jax
import jax.numpy as jnp
from jax.experimental import pallas as pl

N_TARGET = 10000
N_ITEM = 10000
E = 320000


def _ff_body(x_ref, Wi_ref, bi_ref, Wh_ref, bh_ref, Wo_ref, bo_ref, o_ref):
    h = jax.nn.relu(x_ref[...] @ Wi_ref[...] + bi_ref[...])
    h = jax.nn.relu(h @ Wh_ref[...] + bh_ref[...])
    o_ref[...] = h @ Wo_ref[...] + bo_ref[...]


def _ff_pallas(x, Wi, bi, Wh, bh, Wo, bo):
    n = x.shape[0]
    blk = 2000
    grid = n // blk
    return pl.pallas_call(
        _ff_body,
        grid=(grid,),
        in_specs=[
            pl.BlockSpec((blk, x.shape[1]), lambda i: (i, 0)),
            pl.BlockSpec(Wi.shape, lambda i: (0, 0)),
            pl.BlockSpec(bi.shape, lambda i: (0,)),
            pl.BlockSpec(Wh.shape, lambda i: (0, 0)),
            pl.BlockSpec(bh.shape, lambda i: (0,)),
            pl.BlockSpec(Wo.shape, lambda i: (0, 0)),
            pl.BlockSpec(bo.shape, lambda i: (0,)),
        ],
        out_specs=pl.BlockSpec((blk, Wo.shape[1]), lambda i: (i, 0)),
        out_shape=jax.ShapeDtypeStruct((n, Wo.shape[1]), jnp.float32),
    )(x, Wi, bi, Wh, bh, Wo, bo)


def _graph_conv(x_src, src, dst, n_src, n_dst, W, b):
    out_deg = jnp.clip(jnp.zeros((n_src,), jnp.float32).at[src].add(1.0), 1.0, None)
    in_deg = jnp.clip(jnp.zeros((n_dst,), jnp.float32).at[dst].add(1.0), 1.0, None)
    h = x_src * (out_deg ** -0.5)[:, None]
    agg = jax.ops.segment_sum(h[src], dst, num_segments=n_dst)
    agg = agg * (in_deg ** -0.5)[:, None]
    return agg @ W + b


def kernel(input_features, edge_i2t, edge_t2i, embed_item,
           pre_Wi, pre_bi, pre_Wh, pre_bh, pre_Wo, pre_bo,
           c1_W_i2t, c1_b_i2t, c1_W_t2i, c1_b_t2i,
           c2_W_i2t, c2_b_i2t, c2_W_t2i, c2_b_t2i,
           c3_W_i2t, c3_b_i2t, c3_W_t2i, c3_b_t2i,
           post_Wi, post_bi, post_Wh, post_bh, post_Wo, post_bo):
    emb_item = embed_item
    tgt = _ff_pallas(input_features.astype(jnp.float32),
                     pre_Wi, pre_bi, pre_Wh, pre_bh, pre_Wo, pre_bo)
    src_it, dst_it = edge_i2t[0], edge_i2t[1]
    src_ti, dst_ti = edge_t2i[0], edge_t2i[1]
    h1_t = jax.nn.relu(_graph_conv(emb_item, src_it, dst_it, N_ITEM, N_TARGET, c1_W_i2t, c1_b_i2t))
    h1_i = jax.nn.relu(_graph_conv(tgt, src_ti, dst_ti, N_TARGET, N_ITEM, c1_W_t2i, c1_b_t2i))
    h2_t = jax.nn.relu(_graph_conv(h1_i, src_it, dst_it, N_ITEM, N_TARGET, c2_W_i2t, c2_b_i2t))
    h2_i = jax.nn.relu(_graph_conv(h1_t, src_ti, dst_ti, N_TARGET, N_ITEM, c2_W_t2i, c2_b_t2i))
    h3_t = _graph_conv(h2_i, src_it, dst_it, N_ITEM, N_TARGET, c3_W_i2t, c3_b_i2t)
    h3_i = _graph_conv(h2_t, src_ti, dst_ti, N_TARGET, N_ITEM, c3_W_t2i, c3_b_t2i)
    h3_t = _ff_pallas(h3_t, post_Wi, post_bi, post_Wh, post_bh, post_Wo, post_bo)
    return (h3_t, h3_i)


# speedup vs baseline: 3.2010x; 3.2010x over previous
import functools

import jax
import jax.numpy as jnp
from jax import lax
from jax.experimental import pallas as pl
from jax.experimental.pallas import tpu as pltpu
from jax.experimental.pallas import tpu_sc as plsc

N = 10000
E = 320000
NT = 16
E_T = E // NT
CH = 80
NCH = E_T // CH
RT = 640
ACC_R = RT * NT
RT_LAST = N - RT * (NT - 1)

_MESH = plsc.VectorSubcoreMesh(core_axis_name="c", subcore_axis_name="s")
_f32 = jnp.float32


def _deg_kernel(src0, dst0, src1, dst1, ones_hbm, zvec):
    @functools.partial(
        pl.kernel,
        out_type=[jax.ShapeDtypeStruct((N,), _f32) for _ in range(4)],
        mesh=_MESH,
        scratch_types=[
            pltpu.VMEM((CH,), jnp.int32),
            pltpu.VMEM((CH,), _f32),
            pltpu.VMEM_SHARED((N,), _f32),
            pltpu.VMEM_SHARED((N,), _f32),
        ],
    )
    def k(s0, d0, s1, d1, ones_h, zv, o0, o1, o2, o3, idxv, onesv, acca, accb):
        sid = lax.axis_index("s")
        cid = lax.axis_index("c")
        pltpu.sync_copy(ones_h, onesv)

        @pl.when(sid == 0)
        def _():
            pltpu.sync_copy(zv, acca)
            pltpu.sync_copy(zv, accb)

        plsc.subcore_barrier()

        def hist(arr, acc):
            def body(c, carry):
                base = sid * E_T + c * CH
                pltpu.sync_copy(arr.at[pl.ds(base, CH)], idxv)
                pltpu.sync_copy(onesv, acc.at[idxv], add=True)
                return carry
            lax.fori_loop(0, NCH, body, 0)

        @pl.when(cid == 0)
        def _():
            hist(s0, acca)
            hist(d0, accb)

        @pl.when(cid == 1)
        def _():
            hist(s1, acca)
            hist(d1, accb)

        plsc.subcore_barrier()

        @pl.when(jnp.logical_and(sid == 0, cid == 0))
        def _():
            pltpu.sync_copy(acca, o0)
            pltpu.sync_copy(accb, o1)

        @pl.when(jnp.logical_and(sid == 0, cid == 1))
        def _():
            pltpu.sync_copy(acca, o2)
            pltpu.sync_copy(accb, o3)

    return k(src0, dst0, src1, dst1, ones_hbm, zvec)


def _make_agg(D):
    @functools.partial(
        pl.kernel,
        out_type=[jax.ShapeDtypeStruct((N, D), _f32),
                  jax.ShapeDtypeStruct((N, D), _f32)],
        mesh=_MESH,
        scratch_types=[
            pltpu.VMEM((CH,), jnp.int32),
            pltpu.VMEM((CH,), jnp.int32),
            pltpu.VMEM((CH, D), _f32),
            pltpu.VMEM_SHARED((ACC_R, D), _f32),
            pltpu.SemaphoreType.DMA,
        ],
    )
    def k(x0, s0, d0, x1, s1, d1, zrow, out0, out1, sidx, didx, rows, acc, gsem):
        sid = lax.axis_index("s")
        cid = lax.axis_index("c")
        pltpu.sync_copy(zrow, acc.at[pl.ds(sid * RT, RT)])
        plsc.subcore_barrier()

        def run(x, s, d):
            def body(c, carry):
                base = sid * E_T + c * CH
                pltpu.sync_copy(s.at[pl.ds(base, CH)], sidx)
                pltpu.sync_copy(d.at[pl.ds(base, CH)], didx)
                pltpu.async_copy(x.at[sidx], rows, gsem).wait()
                pltpu.sync_copy(rows, acc.at[didx], add=True)
                return carry
            lax.fori_loop(0, NCH, body, 0)

        @pl.when(cid == 0)
        def _():
            run(x0, s0, d0)

        @pl.when(cid == 1)
        def _():
            run(x1, s1, d1)

        plsc.subcore_barrier()

        out = [out0, out1]
        for c in range(2):
            @pl.when(jnp.logical_and(cid == c, sid < NT - 1))
            def _(c=c):
                sl = pl.ds(sid * RT, RT)
                pltpu.sync_copy(acc.at[sl], out[c].at[sl])

            @pl.when(jnp.logical_and(cid == c, sid == NT - 1))
            def _(c=c):
                sl = pl.ds((NT - 1) * RT, RT_LAST)
                pltpu.sync_copy(acc.at[sl], out[c].at[sl])

    return k


_agg128 = _make_agg(128)


def _agg_l3(x64, s0, d0, xe, s1, d1, zrow64, zvec):
    @functools.partial(
        pl.kernel,
        out_type=[jax.ShapeDtypeStruct((N, 64), _f32),
                  jax.ShapeDtypeStruct((N,), _f32)],
        mesh=_MESH,
        scratch_types=[
            pltpu.VMEM((CH,), jnp.int32),
            pltpu.VMEM((CH,), jnp.int32),
            pltpu.VMEM((CH, 64), _f32),
            pltpu.VMEM((CH,), _f32),
            pltpu.VMEM_SHARED((ACC_R, 64), _f32),
            pltpu.VMEM_SHARED((N,), _f32),
            pltpu.SemaphoreType.DMA,
        ],
        compiler_params=pltpu.CompilerParams(use_tc_tiling_on_sc=False),
    )
    def k(x_64, s_0, d_0, x_e, s_1, d_1, zr, zv, out64, oute,
          sidx, didx, rows, vals, acc64, acc1, gsem):
        sid = lax.axis_index("s")
        cid = lax.axis_index("c")

        @pl.when(cid == 0)
        def _():
            pltpu.sync_copy(zr, acc64.at[pl.ds(sid * RT, RT)])

        @pl.when(jnp.logical_and(cid == 1, sid == 0))
        def _():
            pltpu.sync_copy(zv, acc1)

        plsc.subcore_barrier()

        @pl.when(cid == 0)
        def _():
            def body(c, carry):
                base = sid * E_T + c * CH
                pltpu.sync_copy(s_0.at[pl.ds(base, CH)], sidx)
                pltpu.sync_copy(d_0.at[pl.ds(base, CH)], didx)
                pltpu.async_copy(x_64.at[sidx], rows, gsem).wait()
                pltpu.sync_copy(rows, acc64.at[didx], add=True)
                return carry
            lax.fori_loop(0, NCH, body, 0)

        @pl.when(cid == 1)
        def _():
            def body(c, carry):
                base = sid * E_T + c * CH
                pltpu.sync_copy(s_1.at[pl.ds(base, CH)], sidx)
                pltpu.sync_copy(d_1.at[pl.ds(base, CH)], didx)
                pltpu.async_copy(x_e.at[sidx], vals, gsem).wait()
                pltpu.sync_copy(vals, acc1.at[didx], add=True)
                return carry
            lax.fori_loop(0, NCH, body, 0)

        plsc.subcore_barrier()

        @pl.when(jnp.logical_and(cid == 0, sid < NT - 1))
        def _():
            sl = pl.ds(sid * RT, RT)
            pltpu.sync_copy(acc64.at[sl], out64.at[sl])

        @pl.when(jnp.logical_and(cid == 0, sid == NT - 1))
        def _():
            sl = pl.ds((NT - 1) * RT, RT_LAST)
            pltpu.sync_copy(acc64.at[sl], out64.at[sl])

        @pl.when(jnp.logical_and(cid == 1, sid == 0))
        def _():
            pltpu.sync_copy(acc1, oute)

    return k(x64, s0, d0, xe, s1, d1, zrow64, zvec)


_B = 1000
_G = N // _B


def _row_spec(d):
    return pl.BlockSpec((_B, d), lambda i: (i, 0))


def _full_spec(shape):
    if len(shape) == 1:
        return pl.BlockSpec(shape, lambda i: (0,))
    return pl.BlockSpec(shape, lambda i: (0, 0))


def _pre_body(x, emb, so_ti, so_it, Wi, bi, Wh, bh, Wo, bo, y0t, y0i):
    h = jax.nn.relu(x[...] @ Wi[...] + bi[...])
    h = jax.nn.relu(h @ Wh[...] + bh[...])
    t = h @ Wo[...] + bo[...]
    y0t[...] = t * so_ti[...]
    y0i[...] = emb[...] * so_it[...]


def _pre(x, emb, so_ti, so_it, Wi, bi, Wh, bh, Wo, bo):
    return pl.pallas_call(
        _pre_body,
        grid=(_G,),
        in_specs=[_row_spec(256), _row_spec(128), _row_spec(1), _row_spec(1),
                  _full_spec(Wi.shape), _full_spec(bi.shape),
                  _full_spec(Wh.shape), _full_spec(bh.shape),
                  _full_spec(Wo.shape), _full_spec(bo.shape)],
        out_specs=[_row_spec(128), _row_spec(128)],
        out_shape=[jax.ShapeDtypeStruct((N, 128), _f32),
                   jax.ShapeDtypeStruct((N, 128), _f32)],
    )(x, emb, so_ti, so_it, Wi, bi, Wh, bh, Wo, bo)


def _l1_body(aggt, aggi, si_t, si_i, so_it, so_ti, W_it, b_it, W_ti, b_ti,
             y_i, y_t):
    h_t = jax.nn.relu((aggt[...] * si_t[...]) @ W_it[...] + b_it[...])
    y_t[...] = h_t * so_ti[...]
    h_i = jax.nn.relu((aggi[...] * si_i[...]) @ W_ti[...] + b_ti[...])
    y_i[...] = h_i * so_it[...]


def _l1(aggt, aggi, si_t, si_i, so_it, so_ti, W_it, b_it, W_ti, b_ti):
    return pl.pallas_call(
        _l1_body,
        grid=(_G,),
        in_specs=[_row_spec(128), _row_spec(128), _row_spec(1), _row_spec(1),
                  _row_spec(1), _row_spec(1),
                  _full_spec(W_it.shape), _full_spec(b_it.shape),
                  _full_spec(W_ti.shape), _full_spec(b_ti.shape)],
        out_specs=[_row_spec(128), _row_spec(128)],
        out_shape=[jax.ShapeDtypeStruct((N, 128), _f32),
                   jax.ShapeDtypeStruct((N, 128), _f32)],
    )(aggt, aggi, si_t, si_i, so_it, so_ti, W_it, b_it, W_ti, b_ti)


def _l2_body(aggt, aggi, si_t, si_i, so_it, so_ti, W_it, b_it, W_ti, b_ti,
             W3_it, W3_ti, z_it, z_ti):
    h2_t = jax.nn.relu((aggt[...] * si_t[...]) @ W_it[...] + b_it[...])
    h2_i = jax.nn.relu((aggi[...] * si_i[...]) @ W_ti[...] + b_ti[...])
    z_it[...] = (h2_i * so_it[...]) @ W3_it[...]
    z_ti[...] = (h2_t * so_ti[...]) @ W3_ti[...]


def _l2(aggt, aggi, si_t, si_i, so_it, so_ti, W_it, b_it, W_ti, b_ti,
        W3_it, W3_ti):
    return pl.pallas_call(
        _l2_body,
        grid=(_G,),
        in_specs=[_row_spec(128), _row_spec(128), _row_spec(1), _row_spec(1),
                  _row_spec(1), _row_spec(1),
                  _full_spec(W_it.shape), _full_spec(b_it.shape),
                  _full_spec(W_ti.shape), _full_spec(b_ti.shape),
                  _full_spec(W3_it.shape), _full_spec(W3_ti.shape)],
        out_specs=[_row_spec(64), _row_spec(1)],
        out_shape=[jax.ShapeDtypeStruct((N, 64), _f32),
                   jax.ShapeDtypeStruct((N, 1), _f32)],
    )(aggt, aggi, si_t, si_i, so_it, so_ti, W_it, b_it, W_ti, b_ti,
      W3_it, W3_ti)


def _post_body(agg64, agge, si_t, si_i, b3_it, b3_ti, Wi, bi, Wh, bh, Wo, bo,
               o_t, o_i):
    t = agg64[...] * si_t[...] + b3_it[...]
    h = jax.nn.relu(t @ Wi[...] + bi[...])
    h = jax.nn.relu(h @ Wh[...] + bh[...])
    o_t[...] = h @ Wo[...] + bo[...]
    o_i[...] = agge[...] * si_i[...] + b3_ti[...]


def _post(agg64, agge, si_t, si_i, b3_it, b3_ti, Wi, bi, Wh, bh, Wo, bo):
    return pl.pallas_call(
        _post_body,
        grid=(_G,),
        in_specs=[_row_spec(64), _row_spec(1), _row_spec(1), _row_spec(1),
                  _full_spec(b3_it.shape), _full_spec(b3_ti.shape),
                  _full_spec(Wi.shape), _full_spec(bi.shape),
                  _full_spec(Wh.shape), _full_spec(bh.shape),
                  _full_spec(Wo.shape), _full_spec(bo.shape)],
        out_specs=[_row_spec(1), _row_spec(1)],
        out_shape=[jax.ShapeDtypeStruct((N, 1), _f32),
                   jax.ShapeDtypeStruct((N, 1), _f32)],
    )(agg64, agge, si_t, si_i, b3_it, b3_ti, Wi, bi, Wh, bh, Wo, bo)


def kernel(input_features, edge_i2t, edge_t2i, embed_item,
           pre_Wi, pre_bi, pre_Wh, pre_bh, pre_Wo, pre_bo,
           c1_W_i2t, c1_b_i2t, c1_W_t2i, c1_b_t2i,
           c2_W_i2t, c2_b_i2t, c2_W_t2i, c2_b_t2i,
           c3_W_i2t, c3_b_i2t, c3_W_t2i, c3_b_t2i,
           post_Wi, post_bi, post_Wh, post_bh, post_Wo, post_bo):
    src_it, dst_it = edge_i2t[0], edge_i2t[1]
    src_ti, dst_ti = edge_t2i[0], edge_t2i[1]

    ones_hbm = jnp.ones((CH,), _f32)
    zvec = jnp.zeros((N,), _f32)
    zrow128 = jnp.zeros((RT, 128), _f32)
    zrow64 = jnp.zeros((RT, 64), _f32)

    d_out_it, d_in_t, d_out_ti, d_in_i = _deg_kernel(
        src_it, dst_it, src_ti, dst_ti, ones_hbm, zvec)

    def scale(d):
        return (jnp.clip(d, 1.0, None) ** -0.5)[:, None]

    so_it, si_t, so_ti, si_i = map(scale, (d_out_it, d_in_t, d_out_ti, d_in_i))

    y0_t, y0_i = _pre(input_features.astype(_f32), embed_item, so_ti, so_it,
                      pre_Wi, pre_bi, pre_Wh, pre_bh, pre_Wo, pre_bo)

    agg1_t, agg1_i = _agg128(y0_i, src_it, dst_it, y0_t, src_ti, dst_ti, zrow128)

    y1_i, y1_t = _l1(agg1_t, agg1_i, si_t, si_i, so_it, so_ti,
                     c1_W_i2t, c1_b_i2t, c1_W_t2i, c1_b_t2i)

    agg2_t, agg2_i = _agg128(y1_i, src_it, dst_it, y1_t, src_ti, dst_ti, zrow128)

    z_it, z_ti = _l2(agg2_t, agg2_i, si_t, si_i, so_it, so_ti,
                     c2_W_i2t, c2_b_i2t, c2_W_t2i, c2_b_t2i,
                     c3_W_i2t, c3_W_t2i)

    agg3_t, agg3_i = _agg_l3(z_it, src_it, dst_it,
                             z_ti.reshape(N), src_ti, dst_ti, zrow64, zvec)

    h3_t, h3_i = _post(agg3_t, agg3_i.reshape(N, 1), si_t, si_i,
                       c3_b_i2t, c3_b_t2i,
                       post_Wi, post_bi, post_Wh, post_bh, post_Wo, post_bo)
    return (h3_t, h3_i)


# speedup vs baseline: 6.6581x; 2.0800x over previous
import functools

import jax
import jax.numpy as jnp
from jax import lax
from jax.experimental import pallas as pl
from jax.experimental.pallas import tpu as pltpu
from jax.experimental.pallas import tpu_sc as plsc

N = 10000
E = 320000
NT = 16
E_T = E // NT
CH = 80
NCH = E_T // CH
RT = 640
ACC_R = RT * NT
RT_LAST = N - RT * (NT - 1)

_MESH = plsc.VectorSubcoreMesh(core_axis_name="c", subcore_axis_name="s")
_f32 = jnp.float32
R = 5
NBLK = 5
BCH = NCH // NBLK


def _pipe_gather_scatter(x, s4, d4, acc, sid, sidxb, didxb, rows,
                         gsems, ssems):
    for blk in range(NBLK):
        pltpu.sync_copy(s4.at[sid * NBLK + blk], sidxb)
        pltpu.sync_copy(d4.at[sid * NBLK + blk], didxb)
        pltpu.async_copy(x.at[sidxb.at[0]], rows[0], gsems[0])

        def round_body(r, carry):
            for k in range(2):
                b = k
                bo = 1 - k
                c = r * 2 + k
                pltpu.make_async_copy(x.at[sidxb.at[0]], rows[b],
                                      gsems[b]).wait()
                if k == 0:
                    @pl.when(r > 0)
                    def _():
                        pltpu.make_async_copy(rows[bo], acc.at[didxb.at[0]],
                                              ssems[bo]).wait()
                else:
                    pltpu.make_async_copy(rows[bo], acc.at[didxb.at[0]],
                                          ssems[bo]).wait()

                @pl.when(c + 1 < BCH)
                def _(c=c, bo=bo):
                    pltpu.async_copy(x.at[sidxb.at[c + 1]], rows[bo],
                                     gsems[bo])

                pltpu.async_copy(rows[b], acc.at[didxb.at[c]], ssems[b],
                                 add=True)
            return carry

        lax.fori_loop(0, BCH // 2, round_body, 0)
        pltpu.make_async_copy(rows[1], acc.at[didxb.at[0]], ssems[1]).wait()


def _pipe_hist(arr3, hists, sid, idx_all, onesv, ssems):
    pltpu.sync_copy(arr3.at[sid], idx_all)

    def round_body(r, carry):
        for k in range(R):
            c = r * R + k

            @pl.when(r > 0)
            def _(k=k):
                pltpu.make_async_copy(onesv, hists[k].at[idx_all.at[0]],
                                      ssems[k]).wait()

            pltpu.async_copy(onesv, hists[k].at[idx_all.at[c]], ssems[k],
                             add=True)
        return carry

    lax.fori_loop(0, NCH // R, round_body, 0)
    for k in range(R):
        pltpu.make_async_copy(onesv, hists[k].at[idx_all.at[0]],
                              ssems[k]).wait()


def _reduce_hists(hists, out, sid, rbuf, obuf, sz):
    off = sid * 640
    for j in range(R):
        pltpu.sync_copy(hists[j].at[pl.ds(off, sz)],
                        rbuf.at[j, pl.ds(0, sz)])
    for i in range(sz // 16):
        v = rbuf[0, pl.ds(16 * i, 16)]
        for j in range(1, R):
            v = v + rbuf[j, pl.ds(16 * i, 16)]
        obuf[pl.ds(16 * i, 16)] = v
    pltpu.sync_copy(obuf.at[pl.ds(0, sz)], out.at[pl.ds(off, sz)])


def _deg_kernel(src0, dst0, src1, dst1, ones_hbm, zvec):
    @functools.partial(
        pl.kernel,
        out_type=[jax.ShapeDtypeStruct((N,), _f32) for _ in range(4)],
        mesh=_MESH,
        scratch_types=[
            pltpu.VMEM((NCH, CH), jnp.int32),
            pltpu.VMEM((CH,), _f32),
            pltpu.VMEM((R, 640), _f32),
            pltpu.VMEM((640,), _f32),
        ] + [pltpu.VMEM_SHARED((N,), _f32)] * (2 * R)
          + [pltpu.SemaphoreType.DMA] * R,
        compiler_params=pltpu.CompilerParams(use_tc_tiling_on_sc=False),
    )
    def k(s0, d0, s1, d1, ones_h, zv, o0, o1, o2, o3, idx_all, onesv,
          rbuf, obuf, *bufs):
        hista = bufs[:R]
        histb = bufs[R:2 * R]
        ssems = bufs[2 * R:]
        sid = lax.axis_index("s")
        cid = lax.axis_index("c")
        pltpu.sync_copy(ones_h, onesv)

        @pl.when(sid == 0)
        def _():
            for h in hista + histb:
                pltpu.sync_copy(zv, h)

        plsc.subcore_barrier()

        @pl.when(cid == 0)
        def _():
            _pipe_hist(s0, hista, sid, idx_all, onesv, ssems)
            _pipe_hist(d0, histb, sid, idx_all, onesv, ssems)

        @pl.when(cid == 1)
        def _():
            _pipe_hist(s1, hista, sid, idx_all, onesv, ssems)
            _pipe_hist(d1, histb, sid, idx_all, onesv, ssems)

        plsc.subcore_barrier()

        for c, (oa, ob) in enumerate([(o0, o1), (o2, o3)]):
            @pl.when(jnp.logical_and(cid == c, sid < NT - 1))
            def _(oa=oa, ob=ob):
                _reduce_hists(hista, oa, sid, rbuf, obuf, 640)
                _reduce_hists(histb, ob, sid, rbuf, obuf, 640)

            @pl.when(jnp.logical_and(cid == c, sid == NT - 1))
            def _(oa=oa, ob=ob):
                _reduce_hists(hista, oa, sid, rbuf, obuf, 400)
                _reduce_hists(histb, ob, sid, rbuf, obuf, 400)

    return k(src0, dst0, src1, dst1, ones_hbm, zvec)


def _make_agg(D):
    @functools.partial(
        pl.kernel,
        out_type=[jax.ShapeDtypeStruct((N, D), _f32),
                  jax.ShapeDtypeStruct((N, D), _f32)],
        mesh=_MESH,
        scratch_types=[
            pltpu.VMEM((BCH, CH), jnp.int32),
            pltpu.VMEM((BCH, CH), jnp.int32),
            pltpu.VMEM_SHARED((ACC_R, D), _f32),
        ] + [pltpu.VMEM((CH, D), _f32)] * 2
          + [pltpu.SemaphoreType.DMA] * 4,
        compiler_params=pltpu.CompilerParams(use_tc_tiling_on_sc=False),
    )
    def k(x0, s0, d0, x1, s1, d1, zrow, out0, out1, sidx_all, didx_all,
          acc, *bufs):
        rows = bufs[:2]
        gsems = bufs[2:4]
        ssems = bufs[4:6]
        sid = lax.axis_index("s")
        cid = lax.axis_index("c")
        pltpu.sync_copy(zrow, acc.at[pl.ds(sid * RT, RT)])
        plsc.subcore_barrier()

        @pl.when(cid == 0)
        def _():
            _pipe_gather_scatter(x0, s0, d0, acc, sid, sidx_all, didx_all,
                                 rows, gsems, ssems)

        @pl.when(cid == 1)
        def _():
            _pipe_gather_scatter(x1, s1, d1, acc, sid, sidx_all, didx_all,
                                 rows, gsems, ssems)

        plsc.subcore_barrier()

        out = [out0, out1]
        for c in range(2):
            @pl.when(jnp.logical_and(cid == c, sid < NT - 1))
            def _(c=c):
                sl = pl.ds(sid * RT, RT)
                pltpu.sync_copy(acc.at[sl], out[c].at[sl])

            @pl.when(jnp.logical_and(cid == c, sid == NT - 1))
            def _(c=c):
                sl = pl.ds((NT - 1) * RT, RT_LAST)
                pltpu.sync_copy(acc.at[sl], out[c].at[sl])

    return k


_agg128 = _make_agg(128)


def _agg_l3(x64, s0, d0, xe, s1, d1, zrow64, zvec):
    @functools.partial(
        pl.kernel,
        out_type=[jax.ShapeDtypeStruct((N, 64), _f32),
                  jax.ShapeDtypeStruct((N,), _f32)],
        mesh=_MESH,
        scratch_types=[
            pltpu.VMEM((BCH, CH), jnp.int32),
            pltpu.VMEM((BCH, CH), jnp.int32),
            pltpu.VMEM_SHARED((ACC_R, 64), _f32),
            pltpu.VMEM_SHARED((N,), _f32),
        ] + [pltpu.VMEM((CH, 64), _f32)] * 2
          + [pltpu.VMEM((CH,), _f32)] * 2
          + [pltpu.SemaphoreType.DMA] * 4,
        compiler_params=pltpu.CompilerParams(use_tc_tiling_on_sc=False),
    )
    def k(x_64, s_0, d_0, x_e, s_1, d_1, zr, zv, out64, oute,
          sidx_all, didx_all, acc64, acc1, *bufs):
        rows = bufs[:2]
        vals = bufs[2:4]
        gsems = bufs[4:6]
        ssems = bufs[6:8]
        sid = lax.axis_index("s")
        cid = lax.axis_index("c")

        @pl.when(cid == 0)
        def _():
            pltpu.sync_copy(zr, acc64.at[pl.ds(sid * RT, RT)])

        @pl.when(jnp.logical_and(cid == 1, sid == 0))
        def _():
            pltpu.sync_copy(zv, acc1)

        plsc.subcore_barrier()

        @pl.when(cid == 0)
        def _():
            _pipe_gather_scatter(x_64, s_0, d_0, acc64, sid, sidx_all,
                                 didx_all, rows, gsems, ssems)

        @pl.when(cid == 1)
        def _():
            _pipe_gather_scatter(x_e, s_1, d_1, acc1, sid, sidx_all,
                                 didx_all, vals, gsems, ssems)

        plsc.subcore_barrier()

        @pl.when(jnp.logical_and(cid == 0, sid < NT - 1))
        def _():
            sl = pl.ds(sid * RT, RT)
            pltpu.sync_copy(acc64.at[sl], out64.at[sl])

        @pl.when(jnp.logical_and(cid == 0, sid == NT - 1))
        def _():
            sl = pl.ds((NT - 1) * RT, RT_LAST)
            pltpu.sync_copy(acc64.at[sl], out64.at[sl])

        @pl.when(jnp.logical_and(cid == 1, sid == 0))
        def _():
            pltpu.sync_copy(acc1, oute)

    return k(x64, s0, d0, xe, s1, d1, zrow64, zvec)


_B = 1000
_G = N // _B


def _row_spec(d):
    return pl.BlockSpec((_B, d), lambda i: (i, 0))


def _full_spec(shape):
    if len(shape) == 1:
        return pl.BlockSpec(shape, lambda i: (0,))
    return pl.BlockSpec(shape, lambda i: (0, 0))


def _pre_body(x, emb, so_ti, so_it, Wi, bi, Wh, bh, Wo, bo, y0t, y0i):
    h = jax.nn.relu(x[...] @ Wi[...] + bi[...])
    h = jax.nn.relu(h @ Wh[...] + bh[...])
    t = h @ Wo[...] + bo[...]
    y0t[...] = t * so_ti[...]
    y0i[...] = emb[...] * so_it[...]


def _pre(x, emb, so_ti, so_it, Wi, bi, Wh, bh, Wo, bo):
    return pl.pallas_call(
        _pre_body,
        grid=(_G,),
        in_specs=[_row_spec(256), _row_spec(128), _row_spec(1), _row_spec(1),
                  _full_spec(Wi.shape), _full_spec(bi.shape),
                  _full_spec(Wh.shape), _full_spec(bh.shape),
                  _full_spec(Wo.shape), _full_spec(bo.shape)],
        out_specs=[_row_spec(128), _row_spec(128)],
        out_shape=[jax.ShapeDtypeStruct((N, 128), _f32),
                   jax.ShapeDtypeStruct((N, 128), _f32)],
    )(x, emb, so_ti, so_it, Wi, bi, Wh, bh, Wo, bo)


def _l1_body(aggt, aggi, si_t, si_i, so_it, so_ti, W_it, b_it, W_ti, b_ti,
             y_i, y_t):
    h_t = jax.nn.relu((aggt[...] * si_t[...]) @ W_it[...] + b_it[...])
    y_t[...] = h_t * so_ti[...]
    h_i = jax.nn.relu((aggi[...] * si_i[...]) @ W_ti[...] + b_ti[...])
    y_i[...] = h_i * so_it[...]


def _l1(aggt, aggi, si_t, si_i, so_it, so_ti, W_it, b_it, W_ti, b_ti):
    return pl.pallas_call(
        _l1_body,
        grid=(_G,),
        in_specs=[_row_spec(128), _row_spec(128), _row_spec(1), _row_spec(1),
                  _row_spec(1), _row_spec(1),
                  _full_spec(W_it.shape), _full_spec(b_it.shape),
                  _full_spec(W_ti.shape), _full_spec(b_ti.shape)],
        out_specs=[_row_spec(128), _row_spec(128)],
        out_shape=[jax.ShapeDtypeStruct((N, 128), _f32),
                   jax.ShapeDtypeStruct((N, 128), _f32)],
    )(aggt, aggi, si_t, si_i, so_it, so_ti, W_it, b_it, W_ti, b_ti)


def _l2_body(aggt, aggi, si_t, si_i, so_it, so_ti, W_it, b_it, W_ti, b_ti,
             W3_it, W3_ti, z_it, z_ti):
    h2_t = jax.nn.relu((aggt[...] * si_t[...]) @ W_it[...] + b_it[...])
    h2_i = jax.nn.relu((aggi[...] * si_i[...]) @ W_ti[...] + b_ti[...])
    z_it[...] = (h2_i * so_it[...]) @ W3_it[...]
    z_ti[...] = (h2_t * so_ti[...]) @ W3_ti[...]


def _l2(aggt, aggi, si_t, si_i, so_it, so_ti, W_it, b_it, W_ti, b_ti,
        W3_it, W3_ti):
    return pl.pallas_call(
        _l2_body,
        grid=(_G,),
        in_specs=[_row_spec(128), _row_spec(128), _row_spec(1), _row_spec(1),
                  _row_spec(1), _row_spec(1),
                  _full_spec(W_it.shape), _full_spec(b_it.shape),
                  _full_spec(W_ti.shape), _full_spec(b_ti.shape),
                  _full_spec(W3_it.shape), _full_spec(W3_ti.shape)],
        out_specs=[_row_spec(64), _row_spec(1)],
        out_shape=[jax.ShapeDtypeStruct((N, 64), _f32),
                   jax.ShapeDtypeStruct((N, 1), _f32)],
    )(aggt, aggi, si_t, si_i, so_it, so_ti, W_it, b_it, W_ti, b_ti,
      W3_it, W3_ti)


def _post_body(agg64, agge, si_t, si_i, b3_it, b3_ti, Wi, bi, Wh, bh, Wo, bo,
               o_t, o_i):
    t = agg64[...] * si_t[...] + b3_it[...]
    h = jax.nn.relu(t @ Wi[...] + bi[...])
    h = jax.nn.relu(h @ Wh[...] + bh[...])
    o_t[...] = h @ Wo[...] + bo[...]
    o_i[...] = agge[...] * si_i[...] + b3_ti[...]


def _post(agg64, agge, si_t, si_i, b3_it, b3_ti, Wi, bi, Wh, bh, Wo, bo):
    return pl.pallas_call(
        _post_body,
        grid=(_G,),
        in_specs=[_row_spec(64), _row_spec(1), _row_spec(1), _row_spec(1),
                  _full_spec(b3_it.shape), _full_spec(b3_ti.shape),
                  _full_spec(Wi.shape), _full_spec(bi.shape),
                  _full_spec(Wh.shape), _full_spec(bh.shape),
                  _full_spec(Wo.shape), _full_spec(bo.shape)],
        out_specs=[_row_spec(1), _row_spec(1)],
        out_shape=[jax.ShapeDtypeStruct((N, 1), _f32),
                   jax.ShapeDtypeStruct((N, 1), _f32)],
    )(agg64, agge, si_t, si_i, b3_it, b3_ti, Wi, bi, Wh, bh, Wo, bo)


def kernel(input_features, edge_i2t, edge_t2i, embed_item,
           pre_Wi, pre_bi, pre_Wh, pre_bh, pre_Wo, pre_bo,
           c1_W_i2t, c1_b_i2t, c1_W_t2i, c1_b_t2i,
           c2_W_i2t, c2_b_i2t, c2_W_t2i, c2_b_t2i,
           c3_W_i2t, c3_b_i2t, c3_W_t2i, c3_b_t2i,
           post_Wi, post_bi, post_Wh, post_bh, post_Wo, post_bo):
    src_it = edge_i2t[0].reshape(NT * NBLK, BCH, CH)
    dst_it = edge_i2t[1].reshape(NT * NBLK, BCH, CH)
    src_ti = edge_t2i[0].reshape(NT * NBLK, BCH, CH)
    dst_ti = edge_t2i[1].reshape(NT * NBLK, BCH, CH)
    src_it_t = edge_i2t[0].reshape(NT, NCH, CH)
    dst_it_t = edge_i2t[1].reshape(NT, NCH, CH)
    src_ti_t = edge_t2i[0].reshape(NT, NCH, CH)
    dst_ti_t = edge_t2i[1].reshape(NT, NCH, CH)

    ones_hbm = jnp.ones((CH,), _f32)
    zvec = jnp.zeros((N,), _f32)
    zrow128 = jnp.zeros((RT, 128), _f32)
    zrow64 = jnp.zeros((RT, 64), _f32)

    d_out_it, d_in_t, d_out_ti, d_in_i = _deg_kernel(
        src_it_t, dst_it_t, src_ti_t, dst_ti_t, ones_hbm, zvec)

    def scale(d):
        return (jnp.clip(d, 1.0, None) ** -0.5)[:, None]

    so_it, si_t, so_ti, si_i = map(scale, (d_out_it, d_in_t, d_out_ti, d_in_i))

    y0_t, y0_i = _pre(input_features.astype(_f32), embed_item, so_ti, so_it,
                      pre_Wi, pre_bi, pre_Wh, pre_bh, pre_Wo, pre_bo)

    agg1_t, agg1_i = _agg128(y0_i, src_it, dst_it, y0_t, src_ti, dst_ti, zrow128)

    y1_i, y1_t = _l1(agg1_t, agg1_i, si_t, si_i, so_it, so_ti,
                     c1_W_i2t, c1_b_i2t, c1_W_t2i, c1_b_t2i)

    agg2_t, agg2_i = _agg128(y1_i, src_it, dst_it, y1_t, src_ti, dst_ti, zrow128)

    z_it, z_ti = _l2(agg2_t, agg2_i, si_t, si_i, so_it, so_ti,
                     c2_W_i2t, c2_b_i2t, c2_W_t2i, c2_b_t2i,
                     c3_W_i2t, c3_W_t2i)

    agg3_t, agg3_i = _agg_l3(z_it, src_it, dst_it,
                             z_ti.reshape(N), src_ti, dst_ti, zrow64, zvec)

    h3_t, h3_i = _post(agg3_t, agg3_i.reshape(N, 1), si_t, si_i,
                       c3_b_i2t, c3_b_t2i,
                       post_Wi, post_bi, post_Wh, post_bh, post_Wo, post_bo)
    return (h3_t, h3_i)
